# baseline (speedup 1.0000x reference)
import functools

import jax
import jax.numpy as jnp
from jax import lax
from jax.experimental import pallas as pl
from jax.experimental.pallas import tpu as pltpu
from jax.experimental.pallas import tpu_sc as plsc

N_NODES = 10000
N_EDGES = 320000
D = 128

NS = 16
CHUNK = 64
CHUNKS_PW = 316
E_PAD = NS * CHUNKS_PW * CHUNK
N_PAD = 10112
ROWS_PER_SUB = N_PAD // NS
DH = 64
DEGW = 16


def _make_sc_agg(with_deg):
  mesh = plsc.VectorSubcoreMesh(core_axis_name="c", subcore_axis_name="s")

  scratch = (
      [pltpu.VMEM((CHUNKS_PW + 1, CHUNK), jnp.int32)] * 2
      + [pltpu.VMEM((CHUNK, DH), jnp.float32)] * 2
      + [pltpu.VMEM_SHARED((N_PAD, DH), jnp.float32)]
      + [pltpu.SemaphoreType.DMA] * 2
  )
  out_type = [jax.ShapeDtypeStruct((2, N_PAD, DH), jnp.float32)]
  if with_deg:
    scratch += [
        pltpu.VMEM((CHUNK, DEGW), jnp.float32),
        pltpu.VMEM_SHARED((N_PAD, DEGW), jnp.float32),
    ]
    out_type += [jax.ShapeDtypeStruct((2, N_PAD, DEGW), jnp.float32)]

  @functools.partial(
      pl.kernel,
      mesh=mesh,
      out_type=out_type,
      scratch_types=scratch,
      compiler_params=pltpu.CompilerParams(use_tc_tiling_on_sc=False),
  )
  def sc_agg(*args):
    if with_deg:
      (zeros_hbm, zd_hbm, tl_hbm, tr_hbm, src_hbm, dst_hbm,
       out_hbm, dout_hbm,
       srcv, dstv, b0, b1, acc, s0, s1, ones, dacc) = args
    else:
      (zeros_hbm, tl_hbm, tr_hbm, src_hbm, dst_hbm, out_hbm,
       srcv, dstv, b0, b1, acc, s0, s1) = args
    bufs = (b0, b1)
    gsem = (s0, s1)
    cid = lax.axis_index("c")
    sid = lax.axis_index("s")

    pltpu.sync_copy(src_hbm.at[sid], srcv.at[pl.ds(0, CHUNKS_PW)])
    pltpu.sync_copy(dst_hbm.at[sid], dstv.at[pl.ds(0, CHUNKS_PW)])
    pltpu.sync_copy(src_hbm.at[sid, pl.ds(0, 1)],
                    srcv.at[pl.ds(CHUNKS_PW, 1)])

    row0 = sid * ROWS_PER_SUB
    pltpu.sync_copy(zeros_hbm.at[pl.ds(row0, ROWS_PER_SUB)],
                    acc.at[pl.ds(row0, ROWS_PER_SUB)])
    if with_deg:
      pltpu.sync_copy(zd_hbm.at[pl.ds(row0, ROWS_PER_SUB)],
                      dacc.at[pl.ds(row0, ROWS_PER_SUB)])

      one16 = jnp.ones((16,), jnp.float32)

      def fill(i, _):
        ones[i, :] = one16
        return 0

      lax.fori_loop(0, CHUNK, fill, 0)

    plsc.subcore_barrier()

    def start_gather(g, b):
      @pl.when(cid == 0)
      def _():
        pltpu.async_copy(tl_hbm.at[srcv.at[g]], bufs[b], gsem[b])

      @pl.when(cid == 1)
      def _():
        pltpu.async_copy(tr_hbm.at[srcv.at[g]], bufs[b], gsem[b])

    def wait_gather(b):
      pltpu.make_async_copy(tl_hbm.at[srcv.at[0]], bufs[b], gsem[b]).wait()

    start_gather(0, 0)

    def body(t, _):
      g0 = 2 * t
      start_gather(g0 + 1, 1)
      wait_gather(0)
      pltpu.sync_copy(bufs[0], acc.at[dstv.at[g0]], add=True)
      if with_deg:
        @pl.when(cid == 0)
        def _():
          pltpu.sync_copy(ones, dacc.at[dstv.at[g0]], add=True)
      start_gather(g0 + 2, 0)
      wait_gather(1)
      pltpu.sync_copy(bufs[1], acc.at[dstv.at[g0 + 1]], add=True)
      if with_deg:
        @pl.when(cid == 1)
        def _():
          pltpu.sync_copy(ones, dacc.at[dstv.at[g0 + 1]], add=True)
      return 0

    lax.fori_loop(0, CHUNKS_PW // 2, body, 0)
    wait_gather(0)
    plsc.subcore_barrier()

    pltpu.sync_copy(acc.at[pl.ds(row0, ROWS_PER_SUB)],
                    out_hbm.at[cid, pl.ds(row0, ROWS_PER_SUB)])
    if with_deg:
      pltpu.sync_copy(dacc.at[pl.ds(row0, ROWS_PER_SUB)],
                      dout_hbm.at[cid, pl.ds(row0, ROWS_PER_SUB)])

  return sc_agg


_sc_agg_l1 = _make_sc_agg(True)
_sc_agg_l2 = _make_sc_agg(False)

R = 1000


def _tc_layer1(x_ref, p_ref, dp_ref, ws_ref, wn_ref, b_ref,
               hl_ref, hr_ref, r_ref):
  p = p_ref[...]
  agg = jnp.concatenate([p[0], p[1]], axis=1)
  dp = dp_ref[...]
  deg = dp[0][:, :1] + dp[1][:, :1]
  r = 1.0 / jnp.maximum(deg, 1.0)
  h = (jnp.dot(x_ref[...], ws_ref[...], preferred_element_type=jnp.float32)
       + jnp.dot(agg * r, wn_ref[...], preferred_element_type=jnp.float32)
       + b_ref[...])
  h = jnp.maximum(h, 0.0)
  hl_ref[...] = h[:, :DH]
  hr_ref[...] = h[:, DH:]
  r_ref[...] = jnp.broadcast_to(r, (R, 8))


def _tc_layer2(hl_ref, hr_ref, p_ref, r_ref, ws_ref, wn_ref, b_ref, o_ref):
  p = p_ref[...]
  mean = jnp.concatenate([p[0], p[1]], axis=1) * r_ref[:, :1]
  ws = ws_ref[...]
  o_ref[...] = (
      jnp.dot(hl_ref[...], ws[:DH], preferred_element_type=jnp.float32)
      + jnp.dot(hr_ref[...], ws[DH:], preferred_element_type=jnp.float32)
      + jnp.dot(mean, wn_ref[...], preferred_element_type=jnp.float32)
      + b_ref[...])


def _row_block(shape_tail):
  return pl.BlockSpec((R,) + shape_tail, lambda i: (i,) + (0,) * len(shape_tail))


def _part_block(d):
  return pl.BlockSpec((2, R, d), lambda i: (0, i, 0))


def _full_block(shape):
  return pl.BlockSpec(shape, lambda i: (0,) * len(shape))


def kernel(inputs, edge_index, W_self1, W_neigh1, b1, W_self2, W_neigh2, b2):
  x = inputs
  src = edge_index[0].astype(jnp.int32)
  dst = edge_index[1].astype(jnp.int32)
  pad = E_PAD - N_EDGES
  src = jnp.concatenate([src, jnp.zeros((pad,), jnp.int32)])
  dst = jnp.concatenate([dst, jnp.full((pad,), N_NODES, jnp.int32)])
  src3 = src.reshape(NS, CHUNKS_PW, CHUNK)
  dst3 = dst.reshape(NS, CHUNKS_PW, CHUNK)

  xl = x[:, :DH]
  xr = x[:, DH:]
  z1 = jnp.zeros((N_PAD, DH), jnp.float32)
  zd = jnp.zeros((N_PAD, DEGW), jnp.float32)
  p1, dp = _sc_agg_l1(z1, zd, xl, xr, src3, dst3)

  hl, hr, rdeg = pl.pallas_call(
      _tc_layer1,
      grid=(N_NODES // R,),
      in_specs=[
          _row_block((D,)),
          _part_block(DH),
          _part_block(DEGW),
          _full_block((D, D)),
          _full_block((D, D)),
          _full_block((1, D)),
      ],
      out_specs=[_row_block((DH,)), _row_block((DH,)), _row_block((8,))],
      out_shape=[
          jax.ShapeDtypeStruct((N_NODES, DH), jnp.float32),
          jax.ShapeDtypeStruct((N_NODES, DH), jnp.float32),
          jax.ShapeDtypeStruct((N_NODES, 8), jnp.float32),
      ],
  )(x, p1, dp, W_self1, W_neigh1, b1.reshape(1, D))

  (p2,) = _sc_agg_l2(z1, hl, hr, src3, dst3)

  out = pl.pallas_call(
      _tc_layer2,
      grid=(N_NODES // R,),
      in_specs=[
          _row_block((DH,)),
          _row_block((DH,)),
          _part_block(DH),
          _row_block((8,)),
          _full_block((D, D)),
          _full_block((D, D)),
          _full_block((1, D)),
      ],
      out_specs=_row_block((D,)),
      out_shape=jax.ShapeDtypeStruct((N_NODES, D), jnp.float32),
  )(hl, hr, p2, rdeg, W_self2, W_neigh2, b2.reshape(1, D))

  return out


# speedup vs baseline: 1.1535x; 1.1535x over previous
import functools

import jax
import jax.numpy as jnp
from jax import lax
from jax.experimental import pallas as pl
from jax.experimental.pallas import tpu as pltpu
from jax.experimental.pallas import tpu_sc as plsc

N_NODES = 10000
N_EDGES = 320000
D = 128

NS = 16
CHUNK = 128
CHUNKS_PW = 158
E_PAD = NS * CHUNKS_PW * CHUNK
N_PAD = 10112
ROWS_PER_SUB = N_PAD // NS
DH = 64
DEGW = 16


def _make_sc_agg(with_deg):
  mesh = plsc.VectorSubcoreMesh(core_axis_name="c", subcore_axis_name="s")

  scratch = (
      [pltpu.VMEM((CHUNKS_PW + 1, CHUNK), jnp.int32)] * 2
      + [pltpu.VMEM((CHUNK, DH), jnp.float32)] * 2
      + [pltpu.VMEM_SHARED((N_PAD, DH), jnp.float32)]
      + [pltpu.SemaphoreType.DMA] * 2
  )
  out_type = [jax.ShapeDtypeStruct((2, N_PAD, DH), jnp.float32)]
  if with_deg:
    scratch += [
        pltpu.VMEM((CHUNK, DEGW), jnp.float32),
        pltpu.VMEM_SHARED((N_PAD, DEGW), jnp.float32),
    ]
    out_type += [jax.ShapeDtypeStruct((2, N_PAD, DEGW), jnp.float32)]

  @functools.partial(
      pl.kernel,
      mesh=mesh,
      out_type=out_type,
      scratch_types=scratch,
      compiler_params=pltpu.CompilerParams(use_tc_tiling_on_sc=False),
  )
  def sc_agg(*args):
    if with_deg:
      (zeros_hbm, zd_hbm, tl_hbm, tr_hbm, src_hbm, dst_hbm,
       out_hbm, dout_hbm,
       srcv, dstv, b0, b1, acc, s0, s1, ones, dacc) = args
    else:
      (zeros_hbm, tl_hbm, tr_hbm, src_hbm, dst_hbm, out_hbm,
       srcv, dstv, b0, b1, acc, s0, s1) = args
    bufs = (b0, b1)
    gsem = (s0, s1)
    cid = lax.axis_index("c")
    sid = lax.axis_index("s")

    pltpu.sync_copy(src_hbm.at[sid], srcv.at[pl.ds(0, CHUNKS_PW)])
    pltpu.sync_copy(dst_hbm.at[sid], dstv.at[pl.ds(0, CHUNKS_PW)])
    pltpu.sync_copy(src_hbm.at[sid, pl.ds(0, 1)],
                    srcv.at[pl.ds(CHUNKS_PW, 1)])

    row0 = sid * ROWS_PER_SUB
    pltpu.sync_copy(zeros_hbm.at[pl.ds(row0, ROWS_PER_SUB)],
                    acc.at[pl.ds(row0, ROWS_PER_SUB)])
    if with_deg:
      pltpu.sync_copy(zd_hbm.at[pl.ds(row0, ROWS_PER_SUB)],
                      dacc.at[pl.ds(row0, ROWS_PER_SUB)])

      one16 = jnp.ones((16,), jnp.float32)

      def fill(i, _):
        ones[i, :] = one16
        return 0

      lax.fori_loop(0, CHUNK, fill, 0)

    plsc.subcore_barrier()

    def start_gather(g, b):
      @pl.when(cid == 0)
      def _():
        pltpu.async_copy(tl_hbm.at[srcv.at[g]], bufs[b], gsem[b])

      @pl.when(cid == 1)
      def _():
        pltpu.async_copy(tr_hbm.at[srcv.at[g]], bufs[b], gsem[b])

    def wait_gather(b):
      pltpu.make_async_copy(tl_hbm.at[srcv.at[0]], bufs[b], gsem[b]).wait()

    start_gather(0, 0)

    def body(t, _):
      g0 = 2 * t
      start_gather(g0 + 1, 1)
      wait_gather(0)
      pltpu.sync_copy(bufs[0], acc.at[dstv.at[g0]], add=True)
      if with_deg:
        @pl.when(cid == 0)
        def _():
          pltpu.sync_copy(ones, dacc.at[dstv.at[g0]], add=True)
      start_gather(g0 + 2, 0)
      wait_gather(1)
      pltpu.sync_copy(bufs[1], acc.at[dstv.at[g0 + 1]], add=True)
      if with_deg:
        @pl.when(cid == 1)
        def _():
          pltpu.sync_copy(ones, dacc.at[dstv.at[g0 + 1]], add=True)
      return 0

    lax.fori_loop(0, CHUNKS_PW // 2, body, 0)
    wait_gather(0)
    plsc.subcore_barrier()

    pltpu.sync_copy(acc.at[pl.ds(row0, ROWS_PER_SUB)],
                    out_hbm.at[cid, pl.ds(row0, ROWS_PER_SUB)])
    if with_deg:
      pltpu.sync_copy(dacc.at[pl.ds(row0, ROWS_PER_SUB)],
                      dout_hbm.at[cid, pl.ds(row0, ROWS_PER_SUB)])

  return sc_agg


_sc_agg_l1 = _make_sc_agg(True)
_sc_agg_l2 = _make_sc_agg(False)

R = 1000


def _tc_layer1(x_ref, p_ref, dp_ref, ws_ref, wn_ref, b_ref,
               hl_ref, hr_ref, r_ref):
  p = p_ref[...]
  agg = jnp.concatenate([p[0], p[1]], axis=1)
  dp = dp_ref[...]
  deg = dp[0][:, :1] + dp[1][:, :1]
  r = 1.0 / jnp.maximum(deg, 1.0)
  h = (jnp.dot(x_ref[...], ws_ref[...], preferred_element_type=jnp.float32)
       + jnp.dot(agg * r, wn_ref[...], preferred_element_type=jnp.float32)
       + b_ref[...])
  h = jnp.maximum(h, 0.0)
  hl_ref[...] = h[:, :DH]
  hr_ref[...] = h[:, DH:]
  r_ref[...] = jnp.broadcast_to(r, (R, 8))


def _tc_layer2(hl_ref, hr_ref, p_ref, r_ref, ws_ref, wn_ref, b_ref, o_ref):
  p = p_ref[...]
  mean = jnp.concatenate([p[0], p[1]], axis=1) * r_ref[:, :1]
  ws = ws_ref[...]
  o_ref[...] = (
      jnp.dot(hl_ref[...], ws[:DH], preferred_element_type=jnp.float32)
      + jnp.dot(hr_ref[...], ws[DH:], preferred_element_type=jnp.float32)
      + jnp.dot(mean, wn_ref[...], preferred_element_type=jnp.float32)
      + b_ref[...])


def _row_block(shape_tail):
  return pl.BlockSpec((R,) + shape_tail, lambda i: (i,) + (0,) * len(shape_tail))


def _part_block(d):
  return pl.BlockSpec((2, R, d), lambda i: (0, i, 0))


def _full_block(shape):
  return pl.BlockSpec(shape, lambda i: (0,) * len(shape))


def kernel(inputs, edge_index, W_self1, W_neigh1, b1, W_self2, W_neigh2, b2):
  x = inputs
  src = edge_index[0].astype(jnp.int32)
  dst = edge_index[1].astype(jnp.int32)
  pad = E_PAD - N_EDGES
  src = jnp.concatenate([src, jnp.zeros((pad,), jnp.int32)])
  dst = jnp.concatenate([dst, jnp.full((pad,), N_NODES, jnp.int32)])
  src3 = src.reshape(NS, CHUNKS_PW, CHUNK)
  dst3 = dst.reshape(NS, CHUNKS_PW, CHUNK)

  xl = x[:, :DH]
  xr = x[:, DH:]
  z1 = jnp.zeros((N_PAD, DH), jnp.float32)
  zd = jnp.zeros((N_PAD, DEGW), jnp.float32)
  p1, dp = _sc_agg_l1(z1, zd, xl, xr, src3, dst3)

  hl, hr, rdeg = pl.pallas_call(
      _tc_layer1,
      grid=(N_NODES // R,),
      in_specs=[
          _row_block((D,)),
          _part_block(DH),
          _part_block(DEGW),
          _full_block((D, D)),
          _full_block((D, D)),
          _full_block((1, D)),
      ],
      out_specs=[_row_block((DH,)), _row_block((DH,)), _row_block((8,))],
      out_shape=[
          jax.ShapeDtypeStruct((N_NODES, DH), jnp.float32),
          jax.ShapeDtypeStruct((N_NODES, DH), jnp.float32),
          jax.ShapeDtypeStruct((N_NODES, 8), jnp.float32),
      ],
  )(x, p1, dp, W_self1, W_neigh1, b1.reshape(1, D))

  (p2,) = _sc_agg_l2(z1, hl, hr, src3, dst3)

  out = pl.pallas_call(
      _tc_layer2,
      grid=(N_NODES // R,),
      in_specs=[
          _row_block((DH,)),
          _row_block((DH,)),
          _part_block(DH),
          _row_block((8,)),
          _full_block((D, D)),
          _full_block((D, D)),
          _full_block((1, D)),
      ],
      out_specs=_row_block((D,)),
      out_shape=jax.ShapeDtypeStruct((N_NODES, D), jnp.float32),
  )(hl, hr, p2, rdeg, W_self2, W_neigh2, b2.reshape(1, D))

  return out
